# speedup vs baseline: 4.0350x; 4.0350x over previous
import functools

import numpy as np
import jax
import jax.numpy as jnp
from jax import lax
from jax.experimental import pallas as pl
from jax.experimental.pallas import tpu as pltpu
from jax.experimental.pallas import tpu_sc as plsc

D_MODEL = 1024
MAX_LEN = 8192
BATCH = 4
SEQ_LEN = 4096
FLAT = BATCH * SEQ_LEN
NUM_CORES = 2
NUM_SUBCORES = 16
NW = NUM_CORES * NUM_SUBCORES
PER_W = FLAT // NW
CHUNK = 32
N_CHUNKS = PER_W // CHUNK
LANES = 16


def _sinusoid_pe_np(max_len, d_model):
    pos = np.arange(max_len, dtype=np.float32)[:, None]
    i = np.arange(0, d_model, 2, dtype=np.float32)
    div = np.power(10000.0, i / d_model)
    pe = np.zeros((max_len, d_model), dtype=np.float32)
    pe[:, 0::2] = np.sin(pos / div)
    pe[:, 1::2] = np.cos(pos / div)
    return pe


_PE = _sinusoid_pe_np(MAX_LEN, D_MODEL)[:SEQ_LEN].astype(np.float32)


@functools.partial(
    pl.kernel,
    out_type=jax.ShapeDtypeStruct((FLAT, D_MODEL), jnp.float32),
    mesh=plsc.VectorSubcoreMesh(core_axis_name="c", subcore_axis_name="s"),
    scratch_types=[
        pltpu.VMEM((N_CHUNKS, CHUNK), jnp.int32),
        pltpu.VMEM((CHUNK, D_MODEL), jnp.float32),
        pltpu.VMEM((CHUNK, D_MODEL), jnp.float32),
        pltpu.SemaphoreType.DMA,
        pltpu.SemaphoreType.DMA,
    ],
)
def _emb_kernel(x_hbm, table_hbm, pe_hbm, out_hbm, idx_v, rows_v, pe_v,
                gsem, psem):
    wid = lax.axis_index("s") * NUM_CORES + lax.axis_index("c")
    flat_base = wid * PER_W
    s_base = lax.rem(flat_base, SEQ_LEN)

    pltpu.sync_copy(x_hbm.at[wid], idx_v)

    def chunk_body(j, carry):
        g = pltpu.async_copy(table_hbm.at[idx_v.at[j]], rows_v, gsem)
        p = pltpu.async_copy(
            pe_hbm.at[pl.ds(s_base + j * CHUNK, CHUNK)], pe_v, psem)
        g.wait()
        p.wait()

        def add_row(r, c2):
            for c in range(D_MODEL // LANES):
                sl = pl.ds(c * LANES, LANES)
                rows_v[r, sl] = rows_v[r, sl] + pe_v[r, sl]
            return c2

        lax.fori_loop(0, CHUNK, add_row, 0)
        pltpu.sync_copy(rows_v,
                        out_hbm.at[pl.ds(flat_base + j * CHUNK, CHUNK)])
        return carry

    lax.fori_loop(0, N_CHUNKS, chunk_body, 0)


def kernel(x, tok_table):
    x_grouped = x.reshape(NW, N_CHUNKS, CHUNK)
    pe = jnp.asarray(_PE)
    out = _emb_kernel(x_grouped, tok_table, pe)
    return out.reshape(BATCH, SEQ_LEN, D_MODEL)
